# baseline (speedup 1.0000x reference)
import jax
import jax.numpy as jnp
from jax import lax
from jax.experimental import pallas as pl
from jax.experimental.pallas import tpu as pltpu
from jax.experimental.pallas import tpu_sc as plsc

N = 10000
C = 256
E = 160000
NB = 8
NL = 2

HC = 128
NT = 16
CH = 128
NCHT = 80
EPT = NCHT * CH
EPAD = NT * EPT - E
DB = 5200
NPH = 5248
JUNK = 5240
RPT = NPH // NT


def _sc_segsum_body(x_hbm, src_hbm, dst_hbm, zf_hbm, ones_hbm,
                    tok_hbm, out_hbm, cnt_hbm,
                    sidx_c, didx_c, rows, ones_v, acc_sh, sem):
    del tok_hbm
    c = lax.axis_index("c")
    s = lax.axis_index("s")

    pltpu.sync_copy(ones_hbm, ones_v)
    off = s * RPT

    for p in range(2):
        pltpu.sync_copy(zf_hbm, acc_sh.at[pl.ds(off, RPT)])
        plsc.subcore_barrier()

        def chunk(j, carry):
            pltpu.sync_copy(src_hbm.at[c, s, j], sidx_c)
            pltpu.sync_copy(dst_hbm.at[p, s, j], didx_c)
            pltpu.async_copy(x_hbm.at[sidx_c], rows, sem).wait()
            pltpu.sync_copy(rows, acc_sh.at[didx_c], add=True)
            return carry

        lax.fori_loop(0, NCHT, chunk, 0)
        plsc.subcore_barrier()

        pltpu.sync_copy(acc_sh.at[pl.ds(off, RPT)],
                        out_hbm.at[c, p, pl.ds(off, RPT)])
        plsc.subcore_barrier()

        pltpu.sync_copy(zf_hbm, acc_sh.at[pl.ds(off, RPT)])
        plsc.subcore_barrier()

        def cchunk(j, carry):
            pltpu.sync_copy(dst_hbm.at[p, s, j], didx_c)
            pltpu.sync_copy(ones_v, acc_sh.at[didx_c], add=True)
            return carry

        lax.fori_loop(0, NCHT, cchunk, 0)
        plsc.subcore_barrier()

        @pl.when(c == 0)
        def _():
            pltpu.sync_copy(acc_sh.at[pl.ds(off, RPT)],
                            cnt_hbm.at[p, pl.ds(off, RPT)])
        plsc.subcore_barrier()


@jax.jit
def _sc_segsum(x2, src, dst, zf, ones, tok):
    mesh = plsc.VectorSubcoreMesh(core_axis_name="c", subcore_axis_name="s")
    return pl.kernel(
        _sc_segsum_body,
        out_type=(jax.ShapeDtypeStruct((2, 2, NPH, HC), jnp.float32),
                  jax.ShapeDtypeStruct((2, NPH, HC), jnp.float32)),
        mesh=mesh,
        scratch_types=[
            pltpu.VMEM((CH,), jnp.int32),
            pltpu.VMEM((CH,), jnp.int32),
            pltpu.VMEM((CH, HC), jnp.float32),
            pltpu.VMEM((CH, HC), jnp.float32),
            pltpu.VMEM_SHARED((NPH, HC), jnp.float32),
            pltpu.SemaphoreType.DMA,
        ],
    )(x2, src, dst, zf, ones, tok)


def _basis_body(bases_ref, coeffs_ref, out_ref):
    b = bases_ref[0]
    cf = coeffs_ref[...]
    for e in range(3):
        w = b[0] * cf[e, 0]
        for i in range(1, NB):
            w = w + b[i] * cf[e, i]
        out_ref[0, e] = w


@jax.jit
def _basis_combine(bases, coeffs):
    return pl.pallas_call(
        _basis_body,
        grid=(NL,),
        in_specs=[
            pl.BlockSpec((1, NB, C, C), lambda l: (l, 0, 0, 0)),
            pl.BlockSpec((3, NB), lambda l: (0, 0)),
        ],
        out_specs=pl.BlockSpec((1, 3, C, C), lambda l: (l, 0, 0, 0)),
        out_shape=jax.ShapeDtypeStruct((NL, 3, C, C), jnp.float32),
    )(bases, coeffs)


def _mm(a, w):
    return lax.dot_general(a, w, (((1,), (1,)), ((), ())),
                           preferred_element_type=jnp.float32,
                           precision=lax.Precision.HIGHEST)


def _ln_relu(h, prm):
    mu = jnp.mean(h, axis=1, keepdims=True)
    d = h - mu
    var = jnp.mean(d * d, axis=1, keepdims=True)
    y = d * lax.rsqrt(var + 1e-5) * prm[1:2, :] + prm[2:3, :]
    return jnp.maximum(y, 0.0)


def _mean(lo_ref, hi_ref, cnt_ref):
    a = jnp.concatenate([lo_ref[0, 0], hi_ref[0, 0]], axis=1)
    return a / jnp.maximum(cnt_ref[0, :, :1], 1.0)


def _combine2_body(a0l_ref, a0h_ref, cnt0_ref, a1l_ref, a1h_ref, cnt1_ref,
                   x_ref, w0_ref, w1_ref, prm_ref, out_ref):
    x = x_ref[...]
    prm = prm_ref[...]
    m0 = _mean(a0l_ref, a0h_ref, cnt0_ref)
    m1 = _mean(a1l_ref, a1h_ref, cnt1_ref)
    h = _mm(m0 + x, w0_ref[...]) + _mm(m1 + x, w1_ref[...]) + prm[0:1, :]
    out_ref[...] = _ln_relu(h, prm)


def _combine1_body(al_ref, ah_ref, cnt_ref, x_ref, w_ref, prm_ref, out_ref):
    x = x_ref[...]
    prm = prm_ref[...]
    m = _mean(al_ref, ah_ref, cnt_ref)
    h = _mm(m + x, w_ref[...]) + prm[0:1, :]
    out_ref[...] = _ln_relu(h, prm)


_BLK = 400
_GRID = N // _BLK
_PB = DB // _BLK


def _pass_of(i):
    return i // _PB


def _row_of(i):
    return i - _PB * (i // _PB)


def _row_spec(width):
    return pl.BlockSpec((_BLK, width), lambda i: (i, 0))


def _half_spec(core):
    return pl.BlockSpec((1, 1, _BLK, HC),
                        lambda i, cc=core: (cc, _pass_of(i), _row_of(i), 0))


def _cnt_spec():
    return pl.BlockSpec((1, _BLK, HC), lambda i: (_pass_of(i), _row_of(i), 0))


def _full_spec(shape):
    return pl.BlockSpec(shape, lambda i: tuple(0 for _ in shape))


def _acc_specs():
    return [_half_spec(0), _half_spec(1), _cnt_spec()]


@jax.jit
def _combine2(acc0, cnt0, acc1, cnt1, x, w0, w1, prm):
    return pl.pallas_call(
        _combine2_body,
        grid=(_GRID,),
        in_specs=_acc_specs() + _acc_specs() + [
            _row_spec(C),
            _full_spec((C, C)), _full_spec((C, C)), _full_spec((8, C))],
        out_specs=_row_spec(C),
        out_shape=jax.ShapeDtypeStruct((N, C), jnp.float32),
    )(acc0, acc0, cnt0, acc1, acc1, cnt1, x, w0, w1, prm)


@jax.jit
def _combine1(acc, cnt, x, w, prm):
    return pl.pallas_call(
        _combine1_body,
        grid=(_GRID,),
        in_specs=_acc_specs() + [
            _row_spec(C),
            _full_spec((C, C)), _full_spec((8, C))],
        out_specs=_row_spec(C),
        out_shape=jax.ShapeDtypeStruct((N, C), jnp.float32),
    )(acc, acc, cnt, x, w, prm)


def kernel(x_paper, x_author, edge_index_paper_cites_paper,
           edge_index_author_writes_paper, edge_index_paper_rev_writes_author,
           combination_coeffs, bases, bias_l, ln_w, ln_b):
    edges = [edge_index_paper_cites_paper, edge_index_author_writes_paper,
             edge_index_paper_rev_writes_author]
    spad = jnp.zeros((EPAD,), jnp.int32)
    dpad = jnp.full((EPAD,), N, jnp.int32)

    def remap_src(e0):
        s2 = jnp.concatenate([e0, spad]) * 2
        return jnp.stack([s2, s2 + 1]).reshape(2, NT, NCHT, CH)

    def remap_dst(e1):
        d = jnp.concatenate([e1, dpad])
        p0 = jnp.where(d < DB, d, JUNK)
        p1 = jnp.where(d >= DB, d - DB, JUNK)
        return jnp.stack([p0, p1]).reshape(2, NT, NCHT, CH)

    srcs = [remap_src(e[0]) for e in edges]
    dsts = [remap_dst(e[1]) for e in edges]
    zf = jnp.zeros((RPT, HC), jnp.float32)
    ones = jnp.ones((CH, HC), jnp.float32)

    w_all = _basis_combine(bases, combination_coeffs)

    def prm_pack(bias, w, b):
        return jnp.zeros((8, C), jnp.float32).at[0].set(bias).at[1].set(
            w).at[2].set(b)

    xp, xa = x_paper, x_author
    tok = jnp.zeros((8, 8), jnp.float32)
    for l in range(NL):
        xs = [xp, xa, xp]
        accs, cnts = [], []
        for ei in range(3):
            acc, cnt = _sc_segsum(xs[ei].reshape(2 * N, HC), srcs[ei],
                                  dsts[ei], zf, ones, tok)
            accs.append(acc)
            cnts.append(cnt)
            tok = cnt[0, :8, :8]
        prm_p = prm_pack(bias_l[l, 0] + bias_l[l, 1], ln_w[l, 0], ln_b[l, 0])
        prm_a = prm_pack(bias_l[l, 2], ln_w[l, 1], ln_b[l, 1])
        xp_new = _combine2(accs[0], cnts[0], accs[1], cnts[1], xp,
                           w_all[l, 0], w_all[l, 1], prm_p)
        xa_new = _combine1(accs[2], cnts[2], xa, w_all[l, 2], prm_a)
        xp, xa = xp_new, xa_new
    return (xp, xa)
